# baseline (speedup 1.0000x reference)
import functools

import jax
import jax.numpy as jnp
from jax import lax
from jax.experimental import pallas as pl
from jax.experimental.pallas import tpu as pltpu
from jax.experimental.pallas import tpu_sc as plsc

_N_ITEMS = 1000000
_H = 64
_COLS_PER_BLK = 20480


def _matvec_body(wt_ref, tt_ref, b_ref, out_ref):
    res = jnp.dot(wt_ref[...], tt_ref[...], preferred_element_type=jnp.float32)
    out_ref[...] = res[0] + b_ref[0, 0]


def _table_matvec(tableT, W, b):
    nblk = pl.cdiv(_N_ITEMS, _COLS_PER_BLK)
    out = pl.pallas_call(
        _matvec_body,
        grid=(nblk,),
        in_specs=[
            pl.BlockSpec((1, _H), lambda i: (0, 0)),
            pl.BlockSpec((_H, _COLS_PER_BLK), lambda i: (0, i)),
            pl.BlockSpec((1, 1), lambda i: (0, 0)),
        ],
        out_specs=pl.BlockSpec((_COLS_PER_BLK,), lambda i: (i,)),
        out_shape=jax.ShapeDtypeStruct((_N_ITEMS,), jnp.float32),
    )(W.reshape(1, _H), tableT, b.reshape(1, 1))
    return out


def _make_gather(n_idx):
    nw = 32
    assert n_idx % (8 * nw) == 0
    per_w = n_idx // nw
    n_sub = 4
    sub = per_w // n_sub
    mesh = plsc.VectorSubcoreMesh(core_axis_name="c", subcore_axis_name="s")

    @functools.partial(
        pl.kernel,
        mesh=mesh,
        out_type=jax.ShapeDtypeStruct((n_idx,), jnp.float32),
        scratch_types=[
            pltpu.VMEM((per_w,), jnp.int32),
            pltpu.VMEM((per_w,), jnp.float32),
            pltpu.SemaphoreType.DMA,
        ],
    )
    def _gather(tw_hbm, idx_hbm, out_hbm, idx_v, val_v, sem):
        wid = lax.axis_index("s") * 2 + lax.axis_index("c")
        base = wid * per_w
        pltpu.sync_copy(idx_hbm.at[pl.ds(base, per_w)], idx_v)
        copies = [
            pltpu.async_copy(tw_hbm.at[idx_v.at[pl.ds(j * sub, sub)]],
                             val_v.at[pl.ds(j * sub, sub)], sem)
            for j in range(n_sub)
        ]
        for c in copies:
            c.wait()
        pltpu.sync_copy(val_v, out_hbm.at[pl.ds(base, per_w)])

    return _gather


def kernel(x, table, W, b):
    bsz, hist = x.shape
    tw = _table_matvec(table.T, W, b)
    xf = x.T.reshape(-1).astype(jnp.int32)
    out = _make_gather(bsz * hist)(tw, xf)
    return out.reshape(hist, bsz, 1).transpose(1, 0, 2)


# speedup vs baseline: 1.0529x; 1.0529x over previous
import functools

import jax
import jax.numpy as jnp
from jax import lax
from jax.experimental import pallas as pl
from jax.experimental.pallas import tpu as pltpu
from jax.experimental.pallas import tpu_sc as plsc

_N_ITEMS = 1000000
_H = 64
_COLS_PER_BLK = 40960


def _matvec_body(wt_ref, tt_ref, b_ref, out_ref):
    res = jnp.dot(wt_ref[...], tt_ref[...], preferred_element_type=jnp.float32)
    out_ref[...] = res[0] + b_ref[0, 0]


def _table_matvec(tableT, W, b):
    nblk = pl.cdiv(_N_ITEMS, _COLS_PER_BLK)
    out = pl.pallas_call(
        _matvec_body,
        grid=(nblk,),
        in_specs=[
            pl.BlockSpec((1, _H), lambda i: (0, 0)),
            pl.BlockSpec((_H, _COLS_PER_BLK), lambda i: (0, i)),
            pl.BlockSpec((1, 1), lambda i: (0, 0)),
        ],
        out_specs=pl.BlockSpec((_COLS_PER_BLK,), lambda i: (i,)),
        out_shape=jax.ShapeDtypeStruct((_N_ITEMS,), jnp.float32),
    )(W.reshape(1, _H), tableT, b.reshape(1, 1))
    return out


def _make_gather(hist, bsz):
    n_idx = hist * bsz
    nw = 32
    per_w = n_idx // nw
    piece = 1024
    n_piece = per_w // piece
    assert n_idx % nw == 0 and per_w % piece == 0 and bsz % piece == 0
    mesh = plsc.VectorSubcoreMesh(core_axis_name="c", subcore_axis_name="s")

    @functools.partial(
        pl.kernel,
        mesh=mesh,
        out_type=jax.ShapeDtypeStruct((n_idx,), jnp.float32),
        scratch_types=[
            pltpu.VMEM((per_w,), jnp.int32),
            pltpu.VMEM((per_w,), jnp.float32),
            pltpu.SemaphoreType.DMA,
        ],
    )
    def _gather(tw_hbm, x_hbm, out_hbm, idx_v, val_v, sem):
        wid = lax.axis_index("s") * 2 + lax.axis_index("c")
        base = wid * per_w
        copies = []
        for p in range(n_piece):
            flat = base + p * piece
            r = flat // bsz
            c = flat % bsz
            copies.append(pltpu.async_copy(
                x_hbm.at[r, pl.ds(c, piece)],
                idx_v.at[pl.ds(p * piece, piece)], sem))
        for cp in copies:
            cp.wait()
        pltpu.async_copy(tw_hbm.at[idx_v], val_v, sem).wait()
        pltpu.sync_copy(val_v, out_hbm.at[pl.ds(base, per_w)])

    return _gather


def kernel(x, table, W, b):
    bsz, hist = x.shape
    tw = _table_matvec(table.T, W, b)
    out = _make_gather(hist, bsz)(tw, x.T.astype(jnp.int32))
    return out.reshape(hist, bsz, 1).transpose(1, 0, 2)
